# baseline (speedup 1.0000x reference)
import functools

import jax
import jax.numpy as jnp
from jax.experimental import pallas as pl


def _edge_kernel(utb_ref, utf_ref, w0_ref, w1_ref, b0_ref, b1_ref, s_ref,
                 eit_ref, y_ref, ei3_ref, *, blk_e, seq_len):
    i = pl.program_id(0)
    utb = utb_ref[...]
    s = s_ref[0, 0]
    for k, (w_ref, bias_ref) in enumerate(((w0_ref, b0_ref),
                                           (w1_ref, b1_ref))):
        zt = jax.lax.dot_general(
            w_ref[...].astype(jnp.bfloat16), utb,
            (((1,), (0,)), ((), ())),
            preferred_element_type=jnp.float32,
        )
        dyn = jax.nn.sigmoid(zt + bias_ref[:, 0][:, None])
        ut_blk = utf_ref[pl.ds((2 * i + k) * blk_e, blk_e), :]
        yv = s * ut_blk + (1.0 - s) * dyn
        batch = yv.shape[1]
        for bb in range(batch):
            y_ref[bb, pl.ds(k * blk_e, blk_e), :] = jnp.broadcast_to(
                yv[:, bb:bb + 1], (blk_e, seq_len))
    eit = eit_ref[...]
    for c in range(2):
        ei3_ref[c] = jnp.broadcast_to(eit[:, c:c + 1],
                                      (eit.shape[0], seq_len))


def kernel(hidden_states, edge_index, edge_weight, W, b, skip_param):
    seq_len = hidden_states.shape[1]
    E = W.shape[0]
    BE = edge_weight.shape[0]
    batch = BE // E

    ut = edge_weight.reshape(batch, E).T
    utb = ut.astype(jnp.bfloat16)
    b2 = b.reshape(E, 1)
    s2 = skip_param.reshape(1, 1)
    eit = edge_index.T

    blk_e = 256
    sup = 2 * blk_e
    n_blk = E // sup
    blk_i = BE // n_blk

    body = functools.partial(_edge_kernel, blk_e=blk_e, seq_len=seq_len)

    y3, ei3t = pl.pallas_call(
        body,
        grid=(n_blk,),
        in_specs=[
            pl.BlockSpec((E, batch), lambda i: (0, 0)),
            pl.BlockSpec((E, batch), lambda i: (0, 0)),
            pl.BlockSpec((blk_e, E), lambda i: (2 * i, 0)),
            pl.BlockSpec((blk_e, E), lambda i: (2 * i + 1, 0)),
            pl.BlockSpec((blk_e, 1), lambda i: (2 * i, 0)),
            pl.BlockSpec((blk_e, 1), lambda i: (2 * i + 1, 0)),
            pl.BlockSpec((1, 1), lambda i: (0, 0)),
            pl.BlockSpec((blk_i, 2), lambda i: (i, 0)),
        ],
        out_specs=[
            pl.BlockSpec((batch, sup, seq_len), lambda i: (0, i, 0)),
            pl.BlockSpec((2, blk_i, seq_len), lambda i: (0, i, 0)),
        ],
        out_shape=[
            jax.ShapeDtypeStruct((batch, E, seq_len), jnp.float32),
            jax.ShapeDtypeStruct((2, BE, seq_len), jnp.int32),
        ],
    )(utb, ut, W, W, b2, b2, s2, eit)

    out = y3.reshape(BE, seq_len)
    return ei3t, out


# speedup vs baseline: 1.5484x; 1.5484x over previous
import functools

import jax
import jax.numpy as jnp
from jax.experimental import pallas as pl


def _edge_kernel(u_ref, w0_ref, w1_ref, b0_ref, b1_ref, s_ref, y_ref, *,
                 blk_e):
    i = pl.program_id(0)
    u = u_ref[...]
    ub = u.astype(jnp.bfloat16)
    s = s_ref[0, 0]
    for k, (w_ref, bias_ref) in enumerate(((w0_ref, b0_ref),
                                           (w1_ref, b1_ref))):
        z = jax.lax.dot_general(
            ub, w_ref[...].astype(jnp.bfloat16),
            (((1,), (1,)), ((), ())),
            preferred_element_type=jnp.float32,
        )
        dyn = jax.nn.sigmoid(z + bias_ref[0, :][None, :])
        u_blk = u_ref[:, pl.ds((2 * i + k) * blk_e, blk_e)]
        y_ref[:, pl.ds(k * blk_e, blk_e)] = s * u_blk + (1.0 - s) * dyn


def kernel(hidden_states, edge_index, edge_weight, W, b, skip_param):
    seq_len = hidden_states.shape[1]
    E = W.shape[0]
    BE = edge_weight.shape[0]
    batch = BE // E

    u = edge_weight.reshape(batch, E)
    b2 = b.reshape(1, E)
    s2 = skip_param.reshape(1, 1)

    blk_e = 256
    sup = 2 * blk_e
    n_blk = E // sup

    body = functools.partial(_edge_kernel, blk_e=blk_e)

    y2 = pl.pallas_call(
        body,
        grid=(n_blk,),
        in_specs=[
            pl.BlockSpec((batch, E), lambda i: (0, 0)),
            pl.BlockSpec((blk_e, E), lambda i: (2 * i, 0)),
            pl.BlockSpec((blk_e, E), lambda i: (2 * i + 1, 0)),
            pl.BlockSpec((1, blk_e), lambda i: (0, 2 * i)),
            pl.BlockSpec((1, blk_e), lambda i: (0, 2 * i + 1)),
            pl.BlockSpec((1, 1), lambda i: (0, 0)),
        ],
        out_specs=pl.BlockSpec((batch, sup), lambda i: (0, i)),
        out_shape=jax.ShapeDtypeStruct((batch, E), jnp.float32),
    )(u, W, W, b2, b2, s2)

    ei3 = jnp.broadcast_to(edge_index[:, :, None], (2, BE, seq_len))
    out = jnp.broadcast_to(y2.reshape(BE, 1), (BE, seq_len))
    return ei3, out


# speedup vs baseline: 1.5668x; 1.0119x over previous
import functools

import jax
import jax.numpy as jnp
from jax.experimental import pallas as pl


def _edge_kernel(u_ref, *refs, blk_e, quarter, n_stream):
    w_refs = refs[:n_stream]
    b_refs = refs[n_stream:2 * n_stream]
    s_ref = refs[2 * n_stream]
    y_refs = refs[2 * n_stream + 1:]
    i = pl.program_id(0)
    u = u_ref[...]
    ub = u.astype(jnp.bfloat16)
    s = s_ref[0, 0]
    for k in range(n_stream):
        z = jax.lax.dot_general(
            ub, w_refs[k][...].astype(jnp.bfloat16),
            (((1,), (1,)), ((), ())),
            preferred_element_type=jnp.float32,
        )
        dyn = jax.nn.sigmoid(z + b_refs[k][0, :][None, :])
        u_blk = u_ref[:, pl.ds(k * quarter + i * blk_e, blk_e)]
        y_refs[k][...] = s * u_blk + (1.0 - s) * dyn


def kernel(hidden_states, edge_index, edge_weight, W, b, skip_param):
    seq_len = hidden_states.shape[1]
    E = W.shape[0]
    BE = edge_weight.shape[0]
    batch = BE // E
    half = E // 2

    u = edge_weight.reshape(batch, E)
    b2 = b.reshape(1, E)
    s2 = skip_param.reshape(1, 1)

    n_stream = 4
    quarter = E // n_stream
    blk_e = 128
    n_blk = quarter // blk_e

    body = functools.partial(_edge_kernel, blk_e=blk_e, quarter=quarter,
                             n_stream=n_stream)

    w_specs = [
        pl.BlockSpec((blk_e, E), functools.partial(
            lambda k, i: (i + k * n_blk, 0), k))
        for k in range(n_stream)
    ]
    b_specs = [
        pl.BlockSpec((1, blk_e), functools.partial(
            lambda k, i: (0, i + k * n_blk), k))
        for k in range(n_stream)
    ]

    ys = pl.pallas_call(
        body,
        grid=(n_blk,),
        in_specs=(
            [pl.BlockSpec((batch, E), lambda i: (0, 0))]
            + w_specs + b_specs
            + [pl.BlockSpec((1, 1), lambda i: (0, 0))]
        ),
        out_specs=[pl.BlockSpec((batch, blk_e), lambda i: (0, i))
                   for _ in range(n_stream)],
        out_shape=[jax.ShapeDtypeStruct((batch, quarter), jnp.float32)
                   for _ in range(n_stream)],
    )(u, *([W] * n_stream), *([b2] * n_stream), s2)

    y2 = jnp.concatenate(ys, axis=1)
    ei3 = jnp.broadcast_to(edge_index[:, :, None], (2, BE, seq_len))
    out = jnp.broadcast_to(y2.reshape(BE, 1), (BE, seq_len))
    return ei3, out


# speedup vs baseline: 1.5950x; 1.0180x over previous
import functools

import jax
import jax.numpy as jnp
from jax.experimental import pallas as pl


def _edge_kernel(u_ref, *refs, blk_e, quarter, n_stream):
    w_refs = refs[:n_stream]
    b_refs = refs[n_stream:2 * n_stream]
    s_ref = refs[2 * n_stream]
    y_refs = refs[2 * n_stream + 1:]
    i = pl.program_id(0)
    u = u_ref[...]
    ub = u.astype(jnp.bfloat16)
    s = s_ref[0, 0]
    for k in range(n_stream):
        z = jax.lax.dot_general(
            ub, w_refs[k][...].astype(jnp.bfloat16),
            (((1,), (1,)), ((), ())),
            preferred_element_type=jnp.float32,
        )
        dyn = jax.nn.sigmoid(z + b_refs[k][0, :][None, :])
        u_blk = u_ref[:, pl.ds(k * quarter + i * blk_e, blk_e)]
        y_refs[k][...] = s * u_blk + (1.0 - s) * dyn


def kernel(hidden_states, edge_index, edge_weight, W, b, skip_param):
    seq_len = hidden_states.shape[1]
    E = W.shape[0]
    BE = edge_weight.shape[0]
    batch = BE // E
    half = E // 2

    u = edge_weight.reshape(batch, E)
    b2 = b.reshape(1, E)
    s2 = skip_param.reshape(1, 1)

    n_stream = 2
    quarter = E // n_stream
    blk_e = 256
    n_blk = quarter // blk_e

    body = functools.partial(_edge_kernel, blk_e=blk_e, quarter=quarter,
                             n_stream=n_stream)

    w_specs = [
        pl.BlockSpec((blk_e, E), functools.partial(
            lambda k, i: (i + k * n_blk, 0), k))
        for k in range(n_stream)
    ]
    b_specs = [
        pl.BlockSpec((1, blk_e), functools.partial(
            lambda k, i: (0, i + k * n_blk), k))
        for k in range(n_stream)
    ]

    ys = pl.pallas_call(
        body,
        grid=(n_blk,),
        in_specs=(
            [pl.BlockSpec((batch, E), lambda i: (0, 0))]
            + w_specs + b_specs
            + [pl.BlockSpec((1, 1), lambda i: (0, 0))]
        ),
        out_specs=[pl.BlockSpec((batch, blk_e), lambda i: (0, i))
                   for _ in range(n_stream)],
        out_shape=[jax.ShapeDtypeStruct((batch, quarter), jnp.float32)
                   for _ in range(n_stream)],
    )(u, *([W] * n_stream), *([b2] * n_stream), s2)

    y2 = jnp.concatenate(ys, axis=1)
    ei3 = jnp.broadcast_to(edge_index[:, :, None], (2, BE, seq_len))
    out = jnp.broadcast_to(y2.reshape(BE, 1), (BE, seq_len))
    return ei3, out
